# Initial kernel scaffold; baseline (speedup 1.0000x reference)
import functools

import jax
import jax.numpy as jnp
from jax import lax
from jax.experimental import pallas as pl
from jax.experimental.pallas import tpu as pltpu
from jax.experimental.pallas import tpu_sc as plsc

N_NODES = 10000
N_EDGES = 320000
D = 128
NC = 2
NS = 16
NW = NC * NS
EPW = N_EDGES // NW
CHUNK = 80
NCHUNK = EPW // CHUNK
GROUP = 5
CNT_W = 16
ROWS_PER_SUB = N_NODES // NS


def _sc_aggregate(feats, src2d, dst2d, zeros_agg, zeros_cnt, ones_rows):
    mesh = plsc.VectorSubcoreMesh(core_axis_name="c", subcore_axis_name="s")

    @functools.partial(
        pl.kernel,
        out_type=(
            jax.ShapeDtypeStruct((NC * N_NODES, D), jnp.float32),
            jax.ShapeDtypeStruct((NC * N_NODES, CNT_W), jnp.float32),
        ),
        mesh=mesh,
        scratch_types=[
            pltpu.VMEM((NCHUNK, CHUNK), jnp.int32),
            pltpu.VMEM((NCHUNK, CHUNK), jnp.int32),
            pltpu.VMEM((GROUP, CHUNK, D), jnp.float32),
            pltpu.VMEM((CHUNK, CNT_W), jnp.float32),
            pltpu.VMEM_SHARED((N_NODES, D), jnp.float32),
            pltpu.VMEM_SHARED((N_NODES, CNT_W), jnp.float32),
        ]
        + [pltpu.SemaphoreType.DMA] * GROUP,
    )
    def k(x_hbm, src_hbm, dst_hbm, zagg_hbm, zcnt_hbm, ones_hbm,
          agg_out, cnt_out,
          sidx, didx, rows, ones_v, agg_sh, cnt_sh, *gsems):
        cid = lax.axis_index("c")
        sid = lax.axis_index("s")
        wid = cid * NS + sid

        rbase = sid * ROWS_PER_SUB
        pltpu.sync_copy(zagg_hbm.at[pl.ds(rbase, ROWS_PER_SUB)],
                        agg_sh.at[pl.ds(rbase, ROWS_PER_SUB)])
        pltpu.sync_copy(zcnt_hbm.at[pl.ds(rbase, ROWS_PER_SUB)],
                        cnt_sh.at[pl.ds(rbase, ROWS_PER_SUB)])
        pltpu.sync_copy(ones_hbm, ones_v)
        pltpu.sync_copy(src_hbm.at[wid], sidx)
        pltpu.sync_copy(dst_hbm.at[wid], didx)
        plsc.subcore_barrier()

        @pl.loop(0, NCHUNK, step=GROUP)
        def _(j):
            cps = []
            for b in range(GROUP):
                cps.append(pltpu.async_copy(
                    x_hbm.at[sidx.at[j + b]], rows.at[b], gsems[b]))
            for b in range(GROUP):
                cps[b].wait()
                pltpu.sync_copy(rows.at[b], agg_sh.at[didx.at[j + b]], add=True)
                pltpu.sync_copy(ones_v, cnt_sh.at[sidx.at[j + b]], add=True)

        plsc.subcore_barrier()
        obase = cid * N_NODES + rbase
        pltpu.sync_copy(agg_sh.at[pl.ds(rbase, ROWS_PER_SUB)],
                        agg_out.at[pl.ds(obase, ROWS_PER_SUB)])
        pltpu.sync_copy(cnt_sh.at[pl.ds(rbase, ROWS_PER_SUB)],
                        cnt_out.at[pl.ds(obase, ROWS_PER_SUB)])

    return k(feats, src2d, dst2d, zeros_agg, zeros_cnt, ones_rows)


def _tc_dense_body(x_ref, agg_ref, cnt_ref, w1_ref, b1_ref, w2_ref, b2_ref,
                   out_ref):
    h = x_ref[...] + agg_ref[:N_NODES, :] + agg_ref[N_NODES:, :]
    z = jnp.dot(h, w1_ref[...], preferred_element_type=jnp.float32) + b1_ref[...]
    r = jnp.maximum(z, 0.0)
    w = 1.0 + cnt_ref[:N_NODES, 0:1] + cnt_ref[N_NODES:, 0:1]
    s = jnp.sum(r * w, axis=0, keepdims=True)
    out_ref[...] = (jnp.dot(s, w2_ref[...], preferred_element_type=jnp.float32)
                    + float(N_NODES) * b2_ref[...])


def _tc_dense(feats, agg_partial, cnt_partial, W1, b1, W2, b2):
    return pl.pallas_call(
        _tc_dense_body,
        out_shape=jax.ShapeDtypeStruct((1, D), jnp.float32),
    )(feats, agg_partial, cnt_partial, W1, b1.reshape(1, D), W2,
      b2.reshape(1, D))


def kernel(feats, edge_index, W1, b1, W2, b2):
    ei = edge_index.astype(jnp.int32)
    src2d = ei[0].reshape(NW, NCHUNK, CHUNK)
    dst2d = ei[1].reshape(NW, NCHUNK, CHUNK)
    zeros_agg = jnp.zeros((N_NODES, D), jnp.float32)
    zeros_cnt = jnp.zeros((N_NODES, CNT_W), jnp.float32)
    ones_rows = jnp.zeros((CHUNK, CNT_W), jnp.float32).at[:, 0].set(1.0)
    agg_p, cnt_p = _sc_aggregate(feats, src2d, dst2d, zeros_agg, zeros_cnt,
                                 ones_rows)
    return _tc_dense(feats, agg_p, cnt_p, W1, b1, W2, b2)


# speedup vs baseline: 11.4835x; 11.4835x over previous
import functools

import jax
import jax.numpy as jnp
from jax import lax
from jax.experimental import pallas as pl
from jax.experimental.pallas import tpu as pltpu
from jax.experimental.pallas import tpu_sc as plsc

N_NODES = 10000
N_EDGES = 320000
D = 128
DH = D // 2
NC = 2
NS = 16
NW = NC * NS
EPW = N_EDGES // NS
CHUNK = 80
NCHUNK = EPW // CHUNK
HALF_CHUNKS = NCHUNK // 2
GROUP = 5
CNT_W = 16
NP = 10240
ROWS_PER_SUB = NP // NS
CNT_ROWS = 2 * NP
CNT_PER_SUB = CNT_ROWS // NS


def _sc_aggregate(x_cols, src_all, dst2d, zeros_agg, zeros_cnt, ones_rows):
    mesh = plsc.VectorSubcoreMesh(core_axis_name="c", subcore_axis_name="s")

    @functools.partial(
        pl.kernel,
        out_type=(
            jax.ShapeDtypeStruct((NC * NP, DH), jnp.float32),
            jax.ShapeDtypeStruct((NC * CNT_ROWS, CNT_W), jnp.float32),
        ),
        mesh=mesh,
        scratch_types=[
            pltpu.VMEM((NCHUNK, CHUNK), jnp.int32),
            pltpu.VMEM((NCHUNK, CHUNK), jnp.int32),
            pltpu.VMEM((GROUP, CHUNK, DH), jnp.float32),
            pltpu.VMEM((CHUNK, CNT_W), jnp.float32),
            pltpu.VMEM_SHARED((NP, DH), jnp.float32),
            pltpu.VMEM_SHARED((CNT_ROWS, CNT_W), jnp.float32),
        ]
        + [pltpu.SemaphoreType.DMA] * GROUP,
        compiler_params=pltpu.CompilerParams(use_tc_tiling_on_sc=False),
    )
    def k(x_hbm, src_hbm, dst_hbm, zagg_hbm, zcnt_hbm, ones_hbm,
          agg_out, cnt_out,
          sidx, didx, rows, ones_v, agg_sh, cnt_sh, *gsems):
        cid = lax.axis_index("c")
        sid = lax.axis_index("s")
        wid = cid * NS + sid

        rbase = sid * ROWS_PER_SUB
        cbase = sid * CNT_PER_SUB
        pltpu.sync_copy(zagg_hbm.at[pl.ds(rbase, ROWS_PER_SUB)],
                        agg_sh.at[pl.ds(rbase, ROWS_PER_SUB)])
        pltpu.sync_copy(zcnt_hbm.at[pl.ds(cbase, CNT_PER_SUB)],
                        cnt_sh.at[pl.ds(cbase, CNT_PER_SUB)])
        pltpu.sync_copy(ones_hbm, ones_v)
        pltpu.sync_copy(src_hbm.at[wid], sidx)
        pltpu.sync_copy(dst_hbm.at[sid], didx)
        plsc.subcore_barrier()

        @pl.loop(0, NCHUNK, step=GROUP)
        def _(j):
            cps = []
            for b in range(GROUP):
                cps.append(pltpu.async_copy(
                    x_hbm.at[sidx.at[j + b]], rows.at[b], gsems[b]))
            for b in range(GROUP):
                jb = j + b
                cps[b].wait()
                pltpu.sync_copy(rows.at[b], agg_sh.at[didx.at[jb]], add=True)

                @pl.when((jb < HALF_CHUNKS) == (cid == 0))
                def _():
                    pltpu.sync_copy(ones_v, cnt_sh.at[sidx.at[jb]], add=True)

        plsc.subcore_barrier()
        pltpu.sync_copy(agg_sh.at[pl.ds(rbase, ROWS_PER_SUB)],
                        agg_out.at[pl.ds(cid * NP + rbase, ROWS_PER_SUB)])
        pltpu.sync_copy(cnt_sh.at[pl.ds(cbase, CNT_PER_SUB)],
                        cnt_out.at[pl.ds(cid * CNT_ROWS + cbase, CNT_PER_SUB)])

    return k(x_cols, src_all, dst2d, zeros_agg, zeros_cnt, ones_rows)


def _tc_dense_body(x_ref, agg_ref, cnt_ref, w1_ref, b1_ref, w2_ref, b2_ref,
                   out_ref):
    w1 = w1_ref[...]
    a0 = agg_ref[:N_NODES, :]
    a1 = agg_ref[NP:NP + N_NODES, :]
    z = (jnp.dot(x_ref[...], w1, preferred_element_type=jnp.float32)
         + jnp.dot(a0, w1[:DH, :], preferred_element_type=jnp.float32)
         + jnp.dot(a1, w1[DH:, :], preferred_element_type=jnp.float32)
         + b1_ref[...])
    r = jnp.maximum(z, 0.0)
    w = (1.0 + cnt_ref[:N_NODES, 0:1]
         + cnt_ref[CNT_ROWS + N_NODES:CNT_ROWS + 2 * N_NODES, 0:1])
    s = jnp.sum(r * w, axis=0, keepdims=True)
    out_ref[...] = (jnp.dot(s, w2_ref[...], preferred_element_type=jnp.float32)
                    + float(N_NODES) * b2_ref[...])


def _tc_dense(feats, agg, cnt, W1, b1, W2, b2):
    return pl.pallas_call(
        _tc_dense_body,
        out_shape=jax.ShapeDtypeStruct((1, D), jnp.float32),
    )(feats, agg, cnt, W1, b1.reshape(1, D), W2, b2.reshape(1, D))


def kernel(feats, edge_index, W1, b1, W2, b2):
    ei = edge_index.astype(jnp.int32)
    src2d = ei[0].reshape(NS, NCHUNK, CHUNK)
    dst2d = ei[1].reshape(NS, NCHUNK, CHUNK)
    src_all = jnp.concatenate([src2d, src2d + N_NODES], axis=0)
    x_cols = jnp.concatenate([feats[:, :DH], feats[:, DH:]], axis=0)
    zeros_agg = jnp.zeros((NP, DH), jnp.float32)
    zeros_cnt = jnp.zeros((CNT_ROWS, CNT_W), jnp.float32)
    ones_rows = jnp.zeros((CHUNK, CNT_W), jnp.float32).at[:, 0].set(1.0)
    agg, cnt = _sc_aggregate(x_cols, src_all, dst2d, zeros_agg, zeros_cnt,
                             ones_rows)
    return _tc_dense(feats, agg, cnt, W1, b1, W2, b2)
